# baseline (speedup 1.0000x reference)
import functools

import jax
import jax.numpy as jnp
from jax import lax
from jax.experimental import pallas as pl
from jax.experimental.pallas import tpu as pltpu
from jax.experimental.pallas import tpu_sc as plsc

_GAMMA = 12.0
_B = 16384
_D = 128
_NW = 32
_BPW = _B // _NW
_C = 128
_NCHUNK = _BPW // _C

_mesh = plsc.VectorSubcoreMesh(core_axis_name="c", subcore_axis_name="s")


@functools.partial(
    pl.kernel,
    mesh=_mesh,
    out_type=jax.ShapeDtypeStruct((_B,), jnp.float32),
    compiler_params=pltpu.CompilerParams(needs_layout_passes=False),
    scratch_types=[
        pltpu.VMEM((_C,), jnp.int32),
        pltpu.VMEM((_C,), jnp.int32),
        pltpu.VMEM((_C,), jnp.int32),
        pltpu.VMEM((_C, _D), jnp.float32),
        pltpu.VMEM((_C, _D), jnp.float32),
        pltpu.VMEM((_C, _D), jnp.float32),
        pltpu.VMEM((_BPW,), jnp.float32),
        pltpu.SemaphoreType.DMA,
    ],
)
def _kge_score(hidx_hbm, ridx_hbm, tidx_hbm, ent_hbm, rel_hbm, out_hbm,
               hidx_v, ridx_v, tidx_v, h_v, r_v, t_v, score_v, sem):
    wid = lax.axis_index("s") * 2 + lax.axis_index("c")
    base = pl.multiple_of(wid * _BPW, _BPW)

    def chunk_body(ci, carry):
        off = pl.multiple_of(base + ci * _C, _C)
        pltpu.sync_copy(hidx_hbm.at[pl.ds(off, _C)], hidx_v)
        pltpu.sync_copy(ridx_hbm.at[pl.ds(off, _C)], ridx_v)
        pltpu.sync_copy(tidx_hbm.at[pl.ds(off, _C)], tidx_v)
        cp_h = pltpu.async_copy(ent_hbm.at[hidx_v], h_v, sem)
        cp_r = pltpu.async_copy(rel_hbm.at[ridx_v], r_v, sem)
        cp_t = pltpu.async_copy(ent_hbm.at[tidx_v], t_v, sem)
        cp_h.wait()
        cp_r.wait()
        cp_t.wait()

        def group_body(g, carry2):
            row_idx = g * 16 + lax.iota(jnp.int32, 16)
            acc = jnp.zeros((16,), jnp.float32)
            for p in range(_D):
                col = jnp.full((16,), p, jnp.int32)
                hg = plsc.load_gather(h_v, [row_idx, col])
                rg = plsc.load_gather(r_v, [row_idx, col])
                tg = plsc.load_gather(t_v, [row_idx, col])
                acc = acc + jnp.abs(hg + rg - tg)
            score_v[pl.ds(ci * _C + g * 16, 16)] = _GAMMA - acc
            return carry2

        lax.fori_loop(0, _C // 16, group_body, 0)
        return carry

    lax.fori_loop(0, _NCHUNK, chunk_body, 0)
    pltpu.sync_copy(score_v, out_hbm.at[pl.ds(base, _BPW)])


def kernel(sample, ent_emb, rel_emb, time_emb):
    del time_emb
    hidx = sample[:, 0]
    ridx = sample[:, 1]
    tidx = sample[:, 2]
    score = _kge_score(hidx, ridx, tidx, ent_emb, rel_emb)
    return score[:, None]


# speedup vs baseline: 3.1635x; 3.1635x over previous
import functools

import jax
import jax.numpy as jnp
from jax import lax
from jax.experimental import pallas as pl
from jax.experimental.pallas import tpu as pltpu
from jax.experimental.pallas import tpu_sc as plsc

_GAMMA = 12.0
_B = 16384
_D = 128
_NW = 32
_BPW = _B // _NW
_C = 128
_NCHUNK = _BPW // _C
_NSEG = _D // 16
_PSTRIDE = 17

_mesh = plsc.VectorSubcoreMesh(core_axis_name="c", subcore_axis_name="s")


@functools.partial(
    pl.kernel,
    mesh=_mesh,
    out_type=jax.ShapeDtypeStruct((_B,), jnp.float32),
    compiler_params=pltpu.CompilerParams(needs_layout_passes=False),
    scratch_types=[
        pltpu.VMEM((2, _C), jnp.int32),
        pltpu.VMEM((2, _C), jnp.int32),
        pltpu.VMEM((2, _C), jnp.int32),
        pltpu.VMEM((2, _C, _D), jnp.float32),
        pltpu.VMEM((2, _C, _D), jnp.float32),
        pltpu.VMEM((2, _C, _D), jnp.float32),
        pltpu.VMEM((16 * _PSTRIDE,), jnp.float32),
        pltpu.VMEM((_BPW,), jnp.float32),
        pltpu.SemaphoreType.DMA,
        pltpu.SemaphoreType.DMA,
        pltpu.SemaphoreType.DMA,
        pltpu.SemaphoreType.DMA,
    ],
)
def _kge_score(hidx_hbm, ridx_hbm, tidx_hbm, ent_hbm, rel_hbm, out_hbm,
               hi_v, ri_v, ti_v, h_v, r_v, t_v, p_v, score_v,
               isem0, isem1, gsem0, gsem1):
    wid = lax.axis_index("s") * 2 + lax.axis_index("c")
    base = pl.multiple_of(wid * _BPW, _BPW)
    iota16 = lax.iota(jnp.int32, 16)
    isems = (isem0, isem1)
    gsems = (gsem0, gsem1)

    idx_cp = {}
    gat_cp = {}

    def start_idx(ci):
        slot = ci & 1
        off = pl.multiple_of(base + ci * _C, _C)
        idx_cp[ci] = [
            pltpu.async_copy(src.at[pl.ds(off, _C)], dst.at[slot], isems[slot])
            for src, dst in ((hidx_hbm, hi_v), (ridx_hbm, ri_v),
                             (tidx_hbm, ti_v))
        ]

    def start_gather(ci):
        slot = ci & 1
        for cp in idx_cp[ci]:
            cp.wait()
        gat_cp[ci] = [
            pltpu.async_copy(tab.at[idx.at[slot]], dst.at[slot], gsems[slot])
            for tab, idx, dst in ((ent_hbm, hi_v, h_v), (rel_hbm, ri_v, r_v),
                                  (ent_hbm, ti_v, t_v))
        ]

    def compute(ci):
        slot = ci & 1

        def group_body(g, carry):
            for rr in range(16):
                row = g * 16 + rr
                acc = jnp.zeros((16,), jnp.float32)
                for j in range(_NSEG):
                    hseg = h_v[slot, row, pl.ds(j * 16, 16)]
                    rseg = r_v[slot, row, pl.ds(j * 16, 16)]
                    tseg = t_v[slot, row, pl.ds(j * 16, 16)]
                    acc = acc + jnp.abs(hseg + rseg - tseg)
                plsc.store_scatter(p_v, [iota16 * _PSTRIDE + rr], acc)
            tot = jnp.zeros((16,), jnp.float32)
            for j in range(16):
                tot = tot + plsc.load_gather(p_v, [iota16 + j * _PSTRIDE])
            score_v[pl.ds(ci * _C + g * 16, 16)] = _GAMMA - tot
            return carry

        lax.fori_loop(0, _C // 16, group_body, 0)

    start_idx(0)
    start_gather(0)
    start_idx(1)
    for ci in range(_NCHUNK):
        if ci + 1 < _NCHUNK:
            start_gather(ci + 1)
        for cp in gat_cp[ci]:
            cp.wait()
        if ci + 2 < _NCHUNK:
            start_idx(ci + 2)
        compute(ci)

    pltpu.sync_copy(score_v, out_hbm.at[pl.ds(base, _BPW)])


def kernel(sample, ent_emb, rel_emb, time_emb):
    del time_emb
    hidx = sample[:, 0]
    ridx = sample[:, 1]
    tidx = sample[:, 2]
    score = _kge_score(hidx, ridx, tidx, ent_emb, rel_emb)
    return score[:, None]
